# Initial kernel scaffold; baseline (speedup 1.0000x reference)
import jax
import jax.numpy as jnp
from jax.experimental import pallas as pl


def kernel(title_ids, title_token_ids, item_table, text_table):
    raise NotImplementedError("write your pallas kernel here")


# speedup vs baseline: 13.2811x; 13.2811x over previous
import functools

import jax
import jax.numpy as jnp
from jax import lax
from jax.experimental import pallas as pl
from jax.experimental.pallas import tpu as pltpu
from jax.experimental.pallas import tpu_sc as plsc

B = 16384
L = 20
EMB = 32
NC = 2
NS = 16
NW = NC * NS
BPW = B // NW
C = 128
NCH = BPW // C


def _sc_gather_pool(tid_r, tok_r, item_table, text_table):
    mesh = plsc.VectorSubcoreMesh(core_axis_name="c", subcore_axis_name="s")

    @functools.partial(
        pl.kernel,
        out_type=(
            jax.ShapeDtypeStruct((NW, NCH, C, EMB), jnp.float32),
            jax.ShapeDtypeStruct((NW, NCH, C, EMB), jnp.float32),
        ),
        mesh=mesh,
        compiler_params=pltpu.CompilerParams(use_tc_tiling_on_sc=False),
        scratch_types=[
            pltpu.VMEM((NCH, C), jnp.int32),
            pltpu.VMEM((L, NCH, C), jnp.int32),
            pltpu.VMEM((NCH, C, EMB), jnp.float32),
            pltpu.VMEM((L, C, EMB), jnp.float32),
            pltpu.VMEM((NCH, C, EMB), jnp.float32),
            pltpu.SemaphoreType.DMA,
            pltpu.SemaphoreType.DMA,
        ],
    )
    def k(tid_hbm, tok_hbm, item_hbm, text_hbm, ido_hbm, summ_hbm,
          tid_v, tok_v, item_v, gath_v, acc_v, sem_i, sem_g):
        wid = lax.axis_index("s") * NC + lax.axis_index("c")
        pltpu.sync_copy(tid_hbm.at[wid], tid_v)
        pltpu.sync_copy(tok_hbm.at[wid], tok_v)
        item_cps = [
            pltpu.async_copy(item_hbm.at[tid_v.at[c]], item_v.at[c], sem_i)
            for c in range(NCH)
        ]
        for c in range(NCH):
            cps = [
                pltpu.async_copy(text_hbm.at[tok_v.at[j, c]], gath_v.at[j], sem_g)
                for j in range(L)
            ]
            for cp in cps:
                cp.wait()

            def red(i, _, c=c):
                for h in range(2):
                    sl = pl.ds(h * 16, 16)
                    s = gath_v[0, i, sl]
                    for j in range(1, L):
                        s = s + gath_v[j, i, sl]
                    acc_v[c, i, sl] = s
                return 0

            lax.fori_loop(0, C, red, 0)
        for cp in item_cps:
            cp.wait()
        pltpu.sync_copy(item_v, ido_hbm.at[wid])
        pltpu.sync_copy(acc_v, summ_hbm.at[wid])

    return k(tid_r, tok_r, item_table, text_table)


def _tc_finalize(ido, summ, tok, t0):
    R = 2048

    def body(id_ref, sm_ref, tok_ref, t0_ref, o_ref):
        cnt = jnp.sum((tok_ref[...] != 0).astype(jnp.float32), axis=1,
                      keepdims=True)
        text = (sm_ref[...] - (L - cnt) * t0_ref[...]) / jnp.maximum(cnt, 1.0)
        o_ref[...] = jnp.concatenate([id_ref[...], text], axis=1)

    return pl.pallas_call(
        body,
        out_shape=jax.ShapeDtypeStruct((B, 2 * EMB), jnp.float32),
        grid=(B // R,),
        in_specs=[
            pl.BlockSpec((R, EMB), lambda i: (i, 0)),
            pl.BlockSpec((R, EMB), lambda i: (i, 0)),
            pl.BlockSpec((R, L), lambda i: (i, 0)),
            pl.BlockSpec((1, EMB), lambda i: (0, 0)),
        ],
        out_specs=pl.BlockSpec((R, 2 * EMB), lambda i: (i, 0)),
    )(ido, summ, tok, t0)


def kernel(title_ids, title_token_ids, item_table, text_table):
    tid_r = title_ids.reshape(NW, NCH, C)
    tok_r = title_token_ids.reshape(NW, NCH, C, L).transpose(0, 3, 1, 2)
    ido, summ = _sc_gather_pool(tid_r, tok_r, item_table, text_table)
    t0 = text_table[0:1, :]
    return _tc_finalize(ido.reshape(B, EMB), summ.reshape(B, EMB),
                        title_token_ids, t0)
